# baseline (speedup 1.0000x reference)
import dataclasses
import functools

import jax
import jax.numpy as jnp
from jax import lax
from jax.experimental import pallas as pl
from jax.experimental.pallas import tpu as pltpu
from jax.experimental.pallas import tpu_sc as plsc

_NC = 2
_NS = 16
_NW = _NC * _NS
_LANES = 16
_CHUNK = 64
_NBUF = 4
_PREF = 2
_EPS = 1e-12


def _build_comb(pos_table, type_table):
    maxpos, hidden = pos_table.shape
    types = type_table.shape[0]

    def body(pos_ref, type_ref, o_ref):
        for t in range(types):
            o_ref[t * maxpos:(t + 1) * maxpos, :] = (
                pos_ref[...] + type_ref[t:t + 1, :]
            )

    return pl.pallas_call(
        body,
        out_shape=jax.ShapeDtypeStruct((types * maxpos, hidden), jnp.float32),
    )(pos_table, type_table)


def _sc_embed_layernorm(word_table, pos_table, type_table, wids, pids, tids,
                        ln_weight, ln_bias):
    n = wids.shape[0]
    hidden = word_table.shape[1]
    maxpos, types = pos_table.shape[0], type_table.shape[0]
    rows_per_tile = maxpos // _NS
    per_w = n // _NW
    n_chunks = per_w // _CHUNK
    mesh = plsc.VectorSubcoreMesh(core_axis_name="c", subcore_axis_name="s")
    cp = pltpu.CompilerParams()
    if "needs_layout_passes" in pltpu.CompilerParams.__dataclass_fields__:
        cp = dataclasses.replace(cp, needs_layout_passes=False)

    @functools.partial(
        pl.kernel,
        out_type=jax.ShapeDtypeStruct((n, hidden), jnp.float32),
        mesh=mesh,
        compiler_params=cp,
        scratch_types=(
            [
                pltpu.VMEM((per_w,), jnp.int32),
                pltpu.VMEM((per_w,), jnp.int32),
                pltpu.VMEM((per_w,), jnp.int32),
                pltpu.VMEM((hidden,), jnp.float32),
                pltpu.VMEM((hidden,), jnp.float32),
                pltpu.VMEM((maxpos // _NS, hidden), jnp.float32),
                pltpu.VMEM((types * (maxpos // _NS), hidden),
                           jnp.float32),
                pltpu.VMEM((types, hidden), jnp.float32),
                pltpu.VMEM_SHARED((types * maxpos, hidden),
                                  jnp.float32),
            ]
            + [pltpu.VMEM((_CHUNK, hidden), jnp.float32)
               for _ in range(3 * _NBUF)]
            + [pltpu.SemaphoreType.DMA for _ in range(3 * _NBUF)]
        ),
    )
    def k(word_hbm, pos_hbm, type_hbm, wid_hbm, pid_hbm, tid_hbm, lnw_hbm,
          lnb_hbm, out_hbm, wid_v, cid_v, tid_v, lnw_v, lnb_v, pstage_v,
          cstage_v, type_v, comb_sh, *rest):
        ws = rest[:_NBUF]
        cs = rest[_NBUF:2 * _NBUF]
        os_ = rest[2 * _NBUF:3 * _NBUF]
        sws = rest[3 * _NBUF:4 * _NBUF]
        scs = rest[4 * _NBUF:5 * _NBUF]
        sos = rest[5 * _NBUF:6 * _NBUF]
        w = lax.axis_index("s") * _NC + lax.axis_index("c")
        base = w * per_w

        pltpu.sync_copy(wid_hbm.at[pl.ds(base, per_w)], wid_v)
        pltpu.sync_copy(pid_hbm.at[pl.ds(base, per_w)], cid_v)
        pltpu.sync_copy(tid_hbm.at[pl.ds(base, per_w)], tid_v)
        pltpu.sync_copy(lnw_hbm, lnw_v)
        pltpu.sync_copy(lnb_hbm, lnb_v)
        pltpu.sync_copy(type_hbm, type_v)

        @pl.loop(0, per_w, step=_LANES)
        def _(i):
            s = pl.ds(i, _LANES)
            cid_v[s] = cid_v[s] + tid_v[s] * maxpos

        sid = lax.axis_index("s")
        pbase = sid * rows_per_tile
        pltpu.sync_copy(pos_hbm.at[pl.ds(pbase, rows_per_tile)], pstage_v)

        @pl.loop(0, rows_per_tile)
        def _(r):
            for t in range(types):
                for h in range(hidden // _LANES):
                    s = pl.ds(h * _LANES, _LANES)
                    cstage_v[t * rows_per_tile + r, s] = (
                        pstage_v[r, s] + type_v[t, s])

        for t in range(types):
            pltpu.sync_copy(
                cstage_v.at[pl.ds(t * rows_per_tile, rows_per_tile)],
                comb_sh.at[pl.ds(t * maxpos + pbase, rows_per_tile)])
        plsc.subcore_barrier()

        def gather_copies(chunk, b):
            off = pl.multiple_of(chunk * _CHUNK, _CHUNK)
            return (
                pltpu.make_async_copy(
                    word_hbm.at[wid_v.at[pl.ds(off, _CHUNK)]], ws[b], sws[b]),
                pltpu.make_async_copy(
                    comb_sh.at[cid_v.at[pl.ds(off, _CHUNK)]], cs[b], scs[b]),
            )

        def out_copy(chunk, b):
            off = pl.multiple_of(chunk * _CHUNK, _CHUNK)
            return pltpu.make_async_copy(
                os_[b], out_hbm.at[pl.ds(base + off, _CHUNK)], sos[b])

        nh = hidden // _LANES
        lnw_r = [lnw_v[pl.ds(h * _LANES, _LANES)] for h in range(nh)]
        lnb_r = [lnb_v[pl.ds(h * _LANES, _LANES)] for h in range(nh)]
        inv_h = jnp.float32(1.0 / hidden)

        def ln_token(b, t):
            e = []
            for h in range(nh):
                slc = (t, pl.ds(h * _LANES, _LANES))
                e.append(ws[b][*slc] + cs[b][*slc])
            ss = e
            qq = [v * v for v in e]
            while len(ss) > 1:
                ss = [a + c for a, c in zip(ss[::2], ss[1::2])]
                qq = [a + c for a, c in zip(qq[::2], qq[1::2])]
            mean = jnp.sum(ss[0]) * inv_h
            var = jnp.sum(qq[0]) * inv_h - mean * mean
            x = jnp.full((_LANES,), var + _EPS, dtype=jnp.float32)
            i = lax.bitcast_convert_type(x, jnp.int32)
            i = jnp.int32(0x5F3759DF) - lax.shift_right_logical(i, 1)
            y = lax.bitcast_convert_type(i, jnp.float32)
            for _ in range(3):
                y = y * (1.5 - 0.5 * x * y * y)
            m = jnp.full((_LANES,), mean, dtype=jnp.float32)
            for h in range(nh):
                slc = (t, pl.ds(h * _LANES, _LANES))
                os_[b][*slc] = (e[h] - m) * (y * lnw_r[h]) + lnb_r[h]

        for p in range(_PREF):
            for cp_ in gather_copies(p, p):
                cp_.start()

        @pl.loop(0, n_chunks, step=_NBUF)
        def _(g):
            for b in range(_NBUF):
                kk = g + b
                pb = (b + _PREF) % _NBUF

                @pl.when(kk + _PREF < n_chunks)
                def _():
                    @pl.when(kk >= _NBUF - _PREF)
                    def _():
                        out_copy(kk - (_NBUF - _PREF), pb).wait()
                    for cp_ in gather_copies(kk + _PREF, pb):
                        cp_.start()

                for cp_ in gather_copies(kk, b):
                    cp_.wait()

                @pl.loop(0, _CHUNK, step=2)
                def _(t):
                    ln_token(b, t)
                    ln_token(b, t + 1)

                out_copy(kk, b).start()

        for i in range(_NBUF):
            ch = n_chunks - _NBUF + i
            out_copy(ch, ch % _NBUF).wait()

    return k(word_table, pos_table, type_table, wids, pids, tids,
             ln_weight, ln_bias)


def kernel(input_ids, position_ids, token_type_ids, word_table, pos_table,
           type_table, ln_weight, ln_bias):
    b, l = input_ids.shape
    hidden = word_table.shape[1]
    n = b * l
    wids = input_ids.reshape(n).astype(jnp.int32)
    pids = position_ids.reshape(n).astype(jnp.int32)
    tids = token_type_ids.reshape(n).astype(jnp.int32)

    out = _sc_embed_layernorm(word_table, pos_table, type_table, wids, pids,
                              tids, ln_weight, ln_bias)
    return out.reshape(b, l, hidden)


# speedup vs baseline: 1.0059x; 1.0059x over previous
import dataclasses
import functools

import jax
import jax.numpy as jnp
from jax import lax
from jax.experimental import pallas as pl
from jax.experimental.pallas import tpu as pltpu
from jax.experimental.pallas import tpu_sc as plsc

_NC = 2
_NS = 16
_NW = _NC * _NS
_LANES = 16
_CHUNK = 64
_NBUF = 4
_PREF = 2
_EPS = 1e-12


def _sc_embed_layernorm(word_table, pos_table, type_table, wids, pids, tids,
                        ln_weight, ln_bias):
    n = wids.shape[0]
    hidden = word_table.shape[1]
    maxpos, types = pos_table.shape[0], type_table.shape[0]
    rows_per_tile = maxpos // _NS
    per_w = n // _NW
    n_chunks = per_w // _CHUNK
    mesh = plsc.VectorSubcoreMesh(core_axis_name="c", subcore_axis_name="s")
    cp = pltpu.CompilerParams()
    if "needs_layout_passes" in pltpu.CompilerParams.__dataclass_fields__:
        cp = dataclasses.replace(cp, needs_layout_passes=False)

    @functools.partial(
        pl.kernel,
        out_type=jax.ShapeDtypeStruct((n, hidden), jnp.float32),
        mesh=mesh,
        compiler_params=cp,
        scratch_types=(
            [
                pltpu.VMEM((per_w,), jnp.int32),
                pltpu.VMEM((per_w,), jnp.int32),
                pltpu.VMEM((per_w,), jnp.int32),
                pltpu.VMEM((hidden,), jnp.float32),
                pltpu.VMEM((hidden,), jnp.float32),
                pltpu.VMEM((maxpos // _NS, hidden), jnp.float32),
                pltpu.VMEM((types * (maxpos // _NS), hidden),
                           jnp.float32),
                pltpu.VMEM((types, hidden), jnp.float32),
                pltpu.VMEM_SHARED((types * maxpos, hidden),
                                  jnp.float32),
            ]
            + [pltpu.VMEM((_CHUNK, hidden), jnp.float32)
               for _ in range(3 * _NBUF)]
            + [pltpu.SemaphoreType.DMA for _ in range(3 * _NBUF)]
        ),
    )
    def k(word_hbm, pos_hbm, type_hbm, wid_hbm, pid_hbm, tid_hbm, lnw_hbm,
          lnb_hbm, out_hbm, wid_v, cid_v, tid_v, lnw_v, lnb_v, pstage_v,
          cstage_v, type_v, comb_sh, *rest):
        ws = rest[:_NBUF]
        cs = rest[_NBUF:2 * _NBUF]
        os_ = rest[2 * _NBUF:3 * _NBUF]
        sws = rest[3 * _NBUF:4 * _NBUF]
        scs = rest[4 * _NBUF:5 * _NBUF]
        sos = rest[5 * _NBUF:6 * _NBUF]
        w = lax.axis_index("s") * _NC + lax.axis_index("c")
        base = w * per_w

        pltpu.sync_copy(wid_hbm.at[pl.ds(base, per_w)], wid_v)
        pltpu.sync_copy(pid_hbm.at[pl.ds(base, per_w)], cid_v)
        pltpu.sync_copy(tid_hbm.at[pl.ds(base, per_w)], tid_v)
        pltpu.sync_copy(lnw_hbm, lnw_v)
        pltpu.sync_copy(lnb_hbm, lnb_v)
        pltpu.sync_copy(type_hbm, type_v)

        @pl.loop(0, per_w, step=_LANES)
        def _(i):
            s = pl.ds(i, _LANES)
            cid_v[s] = cid_v[s] + tid_v[s] * maxpos

        sid = lax.axis_index("s")
        pbase = sid * rows_per_tile
        pltpu.sync_copy(pos_hbm.at[pl.ds(pbase, rows_per_tile)], pstage_v)

        @pl.loop(0, rows_per_tile)
        def _(r):
            for t in range(types):
                for h in range(hidden // _LANES):
                    s = pl.ds(h * _LANES, _LANES)
                    cstage_v[t * rows_per_tile + r, s] = (
                        pstage_v[r, s] + type_v[t, s])

        for t in range(types):
            pltpu.sync_copy(
                cstage_v.at[pl.ds(t * rows_per_tile, rows_per_tile)],
                comb_sh.at[pl.ds(t * maxpos + pbase, rows_per_tile)])
        plsc.subcore_barrier()

        def gather_copies(chunk, b):
            off = pl.multiple_of(chunk * _CHUNK, _CHUNK)
            return (
                pltpu.make_async_copy(
                    word_hbm.at[wid_v.at[pl.ds(off, _CHUNK)]], ws[b], sws[b]),
                pltpu.make_async_copy(
                    comb_sh.at[cid_v.at[pl.ds(off, _CHUNK)]], cs[b], scs[b]),
            )

        def out_copy(chunk, b):
            off = pl.multiple_of(chunk * _CHUNK, _CHUNK)
            return pltpu.make_async_copy(
                os_[b], out_hbm.at[pl.ds(base + off, _CHUNK)], sos[b])

        nh = hidden // _LANES
        lnw_r = [lnw_v[pl.ds(h * _LANES, _LANES)] for h in range(nh)]
        lnb_r = [lnb_v[pl.ds(h * _LANES, _LANES)] for h in range(nh)]
        inv_h = jnp.float32(1.0 / hidden)

        def ln_token(b, t):
            e = []
            for h in range(nh):
                slc = (t, pl.ds(h * _LANES, _LANES))
                e.append(ws[b][*slc] + cs[b][*slc])
            acc_s = e[0]
            acc_q = e[0] * e[0]
            for h in range(1, nh):
                acc_s = acc_s + e[h]
                acc_q = acc_q + e[h] * e[h]
            mean = jnp.sum(acc_s) * inv_h
            var = jnp.sum(acc_q) * inv_h - mean * mean
            x = jnp.full((_LANES,), var + _EPS, dtype=jnp.float32)
            i = lax.bitcast_convert_type(x, jnp.int32)
            i = jnp.int32(0x5F3759DF) - lax.shift_right_logical(i, 1)
            y = lax.bitcast_convert_type(i, jnp.float32)
            for _ in range(3):
                y = y * (1.5 - 0.5 * x * y * y)
            m = jnp.full((_LANES,), mean, dtype=jnp.float32)
            for h in range(nh):
                slc = (t, pl.ds(h * _LANES, _LANES))
                os_[b][*slc] = (e[h] - m) * (y * lnw_r[h]) + lnb_r[h]

        for p in range(_PREF):
            for cp_ in gather_copies(p, p):
                cp_.start()

        @pl.loop(0, n_chunks, step=_NBUF)
        def _(g):
            for b in range(_NBUF):
                kk = g + b
                pb = (b + _PREF) % _NBUF

                @pl.when(kk + _PREF < n_chunks)
                def _():
                    @pl.when(kk >= _NBUF - _PREF)
                    def _():
                        out_copy(kk - (_NBUF - _PREF), pb).wait()
                    for cp_ in gather_copies(kk + _PREF, pb):
                        cp_.start()

                for cp_ in gather_copies(kk, b):
                    cp_.wait()

                @pl.loop(0, _CHUNK, step=2)
                def _(t):
                    ln_token(b, t)
                    ln_token(b, t + 1)

                out_copy(kk, b).start()

        for i in range(_NBUF):
            ch = n_chunks - _NBUF + i
            out_copy(ch, ch % _NBUF).wait()

    return k(word_table, pos_table, type_table, wids, pids, tids,
             ln_weight, ln_bias)


def kernel(input_ids, position_ids, token_type_ids, word_table, pos_table,
           type_table, ln_weight, ln_bias):
    b, l = input_ids.shape
    hidden = word_table.shape[1]
    n = b * l
    wids = input_ids.reshape(n).astype(jnp.int32)
    pids = position_ids.reshape(n).astype(jnp.int32)
    tids = token_type_ids.reshape(n).astype(jnp.int32)

    out = _sc_embed_layernorm(word_table, pos_table, type_table, wids, pids,
                              tids, ln_weight, ln_bias)
    return out.reshape(b, l, hidden)
